# baseline (device time: 803420 ns/iter reference)
import jax
import jax.numpy as jnp
from jax import lax
from jax.experimental import pallas as pl
from jax.experimental.pallas import tpu as pltpu

N_DEV = 16
N_ROUNDS = 8


def kernel(x):
    m_per, n = x.shape
    half = m_per // 2
    x16 = x.astype(jnp.bfloat16)

    def body(
        x_ref, out_ref, dummy_ref, comm_r, comm_l, copy_sem,
        send_r, recv_r, send_l, recv_l, copy_r, copy_l,
    ):
        my = lax.axis_index("i")
        left = lax.rem(my - 1 + N_DEV, N_DEV)
        right = lax.rem(my + 1, N_DEV)


        local_copy = pltpu.make_async_copy(
            x_ref, out_ref.at[pl.ds(my * m_per, m_per), :], copy_sem
        )
        local_copy.start()

        pending = []
        for h in range(N_ROUNDS):
            last = h == N_ROUNDS - 1
            rows = half if last else m_per
            src_r = x_ref if h == 0 else comm_r.at[(h - 1) % 2]
            rdma_r = pltpu.make_async_remote_copy(
                src_ref=src_r.at[pl.ds(0, rows), :],
                dst_ref=comm_r.at[h % 2, pl.ds(0, rows), :],
                send_sem=send_r.at[h],
                recv_sem=recv_r.at[h],
                device_id=(right,),
                device_id_type=pl.DeviceIdType.MESH,
            )
            rdma_r.start()

            off = half if last else 0
            src_l = x_ref if h == 0 else comm_l.at[(h - 1) % 2]
            rdma_l = pltpu.make_async_remote_copy(
                src_ref=src_l.at[pl.ds(off, rows), :],
                dst_ref=comm_l.at[h % 2, pl.ds(off, rows), :],
                send_sem=send_l.at[h],
                recv_sem=recv_l.at[h],
                device_id=(left,),
                device_id_type=pl.DeviceIdType.MESH,
            )
            rdma_l.start()
            rdma_l.wait()
            rdma_r.wait()

            c_r = lax.rem(my - h - 1 + N_DEV, N_DEV)
            cp_r = pltpu.make_async_copy(
                comm_r.at[h % 2, pl.ds(0, rows), :],
                out_ref.at[pl.ds(c_r * m_per, rows), :],
                copy_r.at[h],
            )
            cp_r.start()
            c_l = lax.rem(my + h + 1, N_DEV)
            cp_l = pltpu.make_async_copy(
                comm_l.at[h % 2, pl.ds(off, rows), :],
                out_ref.at[pl.ds(c_l * m_per + off, rows), :],
                copy_l.at[h],
            )
            cp_l.start()
            pending += [cp_r, cp_l]

        for cp in pending:
            cp.wait()
        local_copy.wait()

    out, _ = pl.pallas_call(
        body,
        out_shape=[
            jax.ShapeDtypeStruct((N_DEV * m_per, n), jnp.bfloat16),
            jax.ShapeDtypeStruct((8, 128), jnp.bfloat16),
        ],
        in_specs=[pl.BlockSpec(memory_space=pltpu.VMEM)],
        out_specs=[
            pl.BlockSpec(memory_space=pl.ANY),
            pl.BlockSpec(memory_space=pl.ANY),
        ],
        scratch_shapes=[
            pltpu.VMEM((2, m_per, n), jnp.bfloat16),
            pltpu.VMEM((2, m_per, n), jnp.bfloat16),
            pltpu.SemaphoreType.DMA,
            pltpu.SemaphoreType.DMA((N_ROUNDS,)),
            pltpu.SemaphoreType.DMA((N_ROUNDS,)),
            pltpu.SemaphoreType.DMA((N_ROUNDS,)),
            pltpu.SemaphoreType.DMA((N_ROUNDS,)),
            pltpu.SemaphoreType.DMA((N_ROUNDS,)),
            pltpu.SemaphoreType.DMA((N_ROUNDS,)),
        ],
    )(x16)
    return out


# device time: 800321 ns/iter; 1.0039x vs baseline; 1.0039x over previous
import jax
import jax.numpy as jnp
from jax import lax
from jax.experimental import pallas as pl
from jax.experimental.pallas import tpu as pltpu

N_DEV = 16
N_ROUNDS = 8


def kernel(x):
    m_per, n = x.shape
    half = m_per // 2
    x16 = x.astype(jnp.bfloat16)

    def body(
        x_ref, out_ref, dummy_ref, comm_r, comm_l, copy_sem,
        send_r, recv_r, send_l, recv_l, copy_r, copy_l,
    ):
        my = lax.axis_index("i")
        left = lax.rem(my - 1 + N_DEV, N_DEV)
        right = lax.rem(my + 1, N_DEV)

        barrier_sem = pltpu.get_barrier_semaphore()
        for nbr in (left, right):
            pl.semaphore_signal(
                barrier_sem, inc=1,
                device_id=(nbr,), device_id_type=pl.DeviceIdType.MESH,
            )
        pl.semaphore_wait(barrier_sem, 2)

        local_copy = pltpu.make_async_copy(
            x_ref, out_ref.at[pl.ds(my * m_per, m_per), :], copy_sem
        )
        local_copy.start()

        def subs(d, h):
            if h < N_ROUNDS - 1:
                return ((0, 0), (1, half))
            return ((0, 0),) if d == "r" else ((1, half),)

        def start_round(h, rdmas):
            for s, off in subs("r", h):
                src = x_ref if h == 0 else comm_r.at[(h - 1) % 2]
                rdma = pltpu.make_async_remote_copy(
                    src_ref=src.at[pl.ds(off, half), :],
                    dst_ref=comm_r.at[h % 2, pl.ds(off, half), :],
                    send_sem=send_r.at[h, s],
                    recv_sem=recv_r.at[h, s],
                    device_id=(right,),
                    device_id_type=pl.DeviceIdType.MESH,
                )
                rdmas[("r", h, s)] = rdma
            for s, off in subs("l", h):
                src = x_ref if h == 0 else comm_l.at[(h - 1) % 2]
                rdma = pltpu.make_async_remote_copy(
                    src_ref=src.at[pl.ds(off, half), :],
                    dst_ref=comm_l.at[h % 2, pl.ds(off, half), :],
                    send_sem=send_l.at[h, s],
                    recv_sem=recv_l.at[h, s],
                    device_id=(left,),
                    device_id_type=pl.DeviceIdType.MESH,
                )
                rdmas[("l", h, s)] = rdma
            order = ["r", "l"]
            for s in (0, 1):
                for d in order:
                    if (d, h, s) in rdmas:
                        rdmas[(d, h, s)].start()

        def copy_out(d, h, s, off, copies):
            if d == "r":
                c = lax.rem(my - h - 1 + N_DEV, N_DEV)
                cp = pltpu.make_async_copy(
                    comm_r.at[h % 2, pl.ds(off, half), :],
                    out_ref.at[pl.ds(c * m_per + off, half), :],
                    copy_r.at[h, s],
                )
            else:
                c = lax.rem(my + h + 1, N_DEV)
                cp = pltpu.make_async_copy(
                    comm_l.at[h % 2, pl.ds(off, half), :],
                    out_ref.at[pl.ds(c * m_per + off, half), :],
                    copy_l.at[h, s],
                )
            cp.start()
            copies.append(cp)

        rdmas = {}
        copies = []
        start_round(0, rdmas)
        for h in range(1, N_ROUNDS):
            for s, off in subs("r", h):
                rdmas[("r", h - 1, s)].wait_recv()
                copy_out("r", h - 1, s, off, copies)
            for s, off in subs("l", h):
                rdmas[("l", h - 1, s)].wait_recv()
                copy_out("l", h - 1, s, off, copies)
            start_round(h, rdmas)
            if h == N_ROUNDS - 1:
                rdmas[("r", h - 1, 1)].wait_recv()
                copy_out("r", h - 1, 1, half, copies)
                rdmas[("l", h - 1, 0)].wait_recv()
                copy_out("l", h - 1, 0, 0, copies)

        rdmas[("r", N_ROUNDS - 1, 0)].wait_recv()
        copy_out("r", N_ROUNDS - 1, 0, 0, copies)
        rdmas[("l", N_ROUNDS - 1, 1)].wait_recv()
        copy_out("l", N_ROUNDS - 1, 1, half, copies)
        for rdma in rdmas.values():
            rdma.wait_send()
        for cp in copies:
            cp.wait()
        local_copy.wait()

    out, _ = pl.pallas_call(
        body,
        out_shape=[
            jax.ShapeDtypeStruct((N_DEV * m_per, n), jnp.bfloat16),
            jax.ShapeDtypeStruct((8, 128), jnp.bfloat16),
        ],
        in_specs=[pl.BlockSpec(memory_space=pltpu.VMEM)],
        out_specs=[
            pl.BlockSpec(memory_space=pl.ANY),
            pl.BlockSpec(memory_space=pl.ANY),
        ],
        scratch_shapes=[
            pltpu.VMEM((2, m_per, n), jnp.bfloat16),
            pltpu.VMEM((2, m_per, n), jnp.bfloat16),
            pltpu.SemaphoreType.DMA,
            pltpu.SemaphoreType.DMA((N_ROUNDS, 2)),
            pltpu.SemaphoreType.DMA((N_ROUNDS, 2)),
            pltpu.SemaphoreType.DMA((N_ROUNDS, 2)),
            pltpu.SemaphoreType.DMA((N_ROUNDS, 2)),
            pltpu.SemaphoreType.DMA((N_ROUNDS, 2)),
            pltpu.SemaphoreType.DMA((N_ROUNDS, 2)),
        ],
        compiler_params=pltpu.CompilerParams(collective_id=0),
    )(x16)
    return out


# device time: 800283 ns/iter; 1.0039x vs baseline; 1.0000x over previous
import jax
import jax.numpy as jnp
from jax import lax
from jax.experimental import pallas as pl
from jax.experimental.pallas import tpu as pltpu

N_DEV = 16
N_ROUNDS = 8


def kernel(x):
    m_per, n = x.shape
    half = m_per // 2
    x16 = x.astype(jnp.bfloat16)

    def body(
        x_ref, out_ref, dummy_ref, comm_r, comm_l, copy_sem,
        send_r, recv_r, send_l, recv_l, copy_r, copy_l,
    ):
        my = lax.axis_index("i")
        left = lax.rem(my - 1 + N_DEV, N_DEV)
        right = lax.rem(my + 1, N_DEV)

        barrier_sem = pltpu.get_barrier_semaphore()
        for nbr in (left, right):
            pl.semaphore_signal(
                barrier_sem, inc=1,
                device_id=(nbr,), device_id_type=pl.DeviceIdType.MESH,
            )
        pl.semaphore_wait(barrier_sem, 2)

        local_copy = pltpu.make_async_copy(
            x_ref, out_ref.at[pl.ds(my * m_per, m_per), :], copy_sem
        )
        local_copy.start()

        def subs(d, h):
            if h < N_ROUNDS - 1:
                return ((0, 0), (1, half))
            return ((0, 0),) if d == "r" else ((1, half),)

        def start_round(h, rdmas):
            for s, off in subs("r", h):
                src = x_ref if h == 0 else comm_r.at[(h - 1) % 2]
                rdma = pltpu.make_async_remote_copy(
                    src_ref=src.at[pl.ds(off, half), :],
                    dst_ref=comm_r.at[h % 2, pl.ds(off, half), :],
                    send_sem=send_r.at[h, s],
                    recv_sem=recv_r.at[h, s],
                    device_id=(right,),
                    device_id_type=pl.DeviceIdType.MESH,
                )
                rdmas[("r", h, s)] = rdma
            for s, off in subs("l", h):
                src = x_ref if h == 0 else comm_l.at[(h - 1) % 2]
                rdma = pltpu.make_async_remote_copy(
                    src_ref=src.at[pl.ds(off, half), :],
                    dst_ref=comm_l.at[h % 2, pl.ds(off, half), :],
                    send_sem=send_l.at[h, s],
                    recv_sem=recv_l.at[h, s],
                    device_id=(left,),
                    device_id_type=pl.DeviceIdType.MESH,
                )
                rdmas[("l", h, s)] = rdma
            order = ["r", "l"]
            for s in (0, 1):
                for d in order:
                    if (d, h, s) in rdmas:
                        rdmas[(d, h, s)].start()

        def copy_out(d, h, s, off, copies):
            if d == "r":
                c = lax.rem(my - h - 1 + N_DEV, N_DEV)
                cp = pltpu.make_async_copy(
                    comm_r.at[h % 2, pl.ds(off, half), :],
                    out_ref.at[pl.ds(c * m_per + off, half), :],
                    copy_r.at[h, s],
                )
            else:
                c = lax.rem(my + h + 1, N_DEV)
                cp = pltpu.make_async_copy(
                    comm_l.at[h % 2, pl.ds(off, half), :],
                    out_ref.at[pl.ds(c * m_per + off, half), :],
                    copy_l.at[h, s],
                )
            cp.start()
            copies.append(cp)

        rdmas = {}
        copies = []
        start_round(0, rdmas)
        for h in range(1, N_ROUNDS):
            for s, off in subs("r", h):
                rdmas[("r", h - 1, s)].wait_recv()
                copy_out("r", h - 1, s, off, copies)
            for s, off in subs("l", h):
                rdmas[("l", h - 1, s)].wait_recv()
                copy_out("l", h - 1, s, off, copies)
            start_round(h, rdmas)
            if h == N_ROUNDS - 1:
                rdmas[("r", h - 1, 1)].wait_recv()
                copy_out("r", h - 1, 1, half, copies)
                rdmas[("l", h - 1, 0)].wait_recv()
                copy_out("l", h - 1, 0, 0, copies)

        rdmas[("r", N_ROUNDS - 1, 0)].wait_recv()
        copy_out("r", N_ROUNDS - 1, 0, 0, copies)
        rdmas[("l", N_ROUNDS - 1, 1)].wait_recv()
        copy_out("l", N_ROUNDS - 1, 1, half, copies)
        for rdma in rdmas.values():
            rdma.wait_send()
        for cp in copies:
            cp.wait()
        local_copy.wait()

    out, _ = pl.pallas_call(
        body,
        out_shape=[
            jax.ShapeDtypeStruct((N_DEV * m_per, n), jnp.bfloat16),
            jax.ShapeDtypeStruct((8, 128), jnp.bfloat16),
        ],
        in_specs=[pl.BlockSpec(memory_space=pltpu.VMEM)],
        out_specs=[
            pl.BlockSpec(memory_space=pltpu.MemorySpace.HBM),
            pl.BlockSpec(memory_space=pl.ANY),
        ],
        scratch_shapes=[
            pltpu.VMEM((2, m_per, n), jnp.bfloat16),
            pltpu.VMEM((2, m_per, n), jnp.bfloat16),
            pltpu.SemaphoreType.DMA,
            pltpu.SemaphoreType.DMA((N_ROUNDS, 2)),
            pltpu.SemaphoreType.DMA((N_ROUNDS, 2)),
            pltpu.SemaphoreType.DMA((N_ROUNDS, 2)),
            pltpu.SemaphoreType.DMA((N_ROUNDS, 2)),
            pltpu.SemaphoreType.DMA((N_ROUNDS, 2)),
            pltpu.SemaphoreType.DMA((N_ROUNDS, 2)),
        ],
        compiler_params=pltpu.CompilerParams(collective_id=0),
    )(x16)
    return out


# device time: 644551 ns/iter; 1.2465x vs baseline; 1.2416x over previous
import jax
import jax.numpy as jnp
from jax import lax
from jax.experimental import pallas as pl
from jax.experimental.pallas import tpu as pltpu

N_DEV = 16
NZ = 4
KMAX = NZ - 1


def kernel(x):
    m_per, n = x.shape
    half = m_per // 2
    x16 = x.astype(jnp.bfloat16)

    def body(
        x_ref, out_ref, dummy_ref, copy_sem,
        up_s, up_r, dn_s, dn_r,
        fx_own_s, fx_own_r, fy_own_s, fy_own_r,
        fx_up_s, fx_up_r, fx_dn_s, fx_dn_r,
        fy_up_s, fy_up_r, fy_dn_s, fy_dn_r,
        gy_own_s, gy_own_r, hx_own_s, hx_own_r,
        gy_up_s, gy_up_r, gy_dn_s, gy_dn_r,
        hx_up_s, hx_up_r, hx_dn_s, hx_dn_r,
    ):
        my = lax.axis_index("i")
        z = lax.div(my, 4)
        s = lax.rem(my, 4)
        px = 4 * z + jnp.bitwise_xor(s, 1)
        py = 4 * z + (3 - s)
        up = my + 4
        dn = my - 4

        def full(o):
            return out_ref.at[pl.ds(o * m_per, m_per), :]

        def top(o):
            return out_ref.at[pl.ds(o * m_per, half), :]

        def bot(o):
            return out_ref.at[pl.ds(o * m_per + half, half), :]

        def rdma(src, dst, ssem, rsem, target):
            return pltpu.make_async_remote_copy(
                src_ref=src, dst_ref=dst, send_sem=ssem, recv_sem=rsem,
                device_id=(target,), device_id_type=pl.DeviceIdType.MESH,
            )

        barrier_sem = pltpu.get_barrier_semaphore()
        for nbr in (px, py):
            pl.semaphore_signal(
                barrier_sem, inc=1,
                device_id=(nbr,), device_id_type=pl.DeviceIdType.MESH,
            )

        @pl.when(z < NZ - 1)
        def _():
            pl.semaphore_signal(
                barrier_sem, inc=1,
                device_id=(up,), device_id_type=pl.DeviceIdType.MESH,
            )

        @pl.when(z > 0)
        def _():
            pl.semaphore_signal(
                barrier_sem, inc=1,
                device_id=(dn,), device_id_type=pl.DeviceIdType.MESH,
            )

        deg = 2 + (z < NZ - 1).astype(jnp.int32) + (z > 0).astype(jnp.int32)
        pl.semaphore_wait(barrier_sem, deg)

        local_copy = pltpu.make_async_copy(
            x_ref, full(my), copy_sem
        )
        local_copy.start()

        @pl.when(z < NZ - 1)
        def _():
            rdma(x_ref, full(my), up_s.at[0], up_r.at[0], up).start()

        @pl.when(z > 0)
        def _():
            rdma(x_ref, full(my), dn_s.at[0], dn_r.at[0], dn).start()

        rdma(x_ref, full(my), fx_own_s, fx_own_r, px).start()
        rdma(x_ref, full(my), fy_own_s, fy_own_r, py).start()

        for k in range(KMAX):
            @pl.when(z >= k + 1)
            def _(k=k):
                o = my - 4 * (k + 1)
                rdma(full(o), full(o), up_s.at[k], up_r.at[k], up).wait_recv()
                if k < KMAX - 1:
                    @pl.when(z < NZ - 1)
                    def _():
                        rdma(
                            full(o), full(o),
                            up_s.at[k + 1], up_r.at[k + 1], up,
                        ).start()
                rdma(full(o), full(o), fx_up_s.at[k], fx_up_r.at[k], px).start()
                rdma(full(o), full(o), fy_up_s.at[k], fy_up_r.at[k], py).start()

            @pl.when(z <= 2 - k)
            def _(k=k):
                o = my + 4 * (k + 1)
                rdma(full(o), full(o), dn_s.at[k], dn_r.at[k], dn).wait_recv()
                if k < KMAX - 1:
                    @pl.when(z > 0)
                    def _():
                        rdma(
                            full(o), full(o),
                            dn_s.at[k + 1], dn_r.at[k + 1], dn,
                        ).start()
                rdma(full(o), full(o), fx_dn_s.at[k], fx_dn_r.at[k], px).start()
                rdma(full(o), full(o), fy_dn_s.at[k], fy_dn_r.at[k], py).start()

        rdma(full(px), full(px), fx_own_s, fx_own_r, px).wait_recv()
        rdma(top(px), top(px), gy_own_s, gy_own_r, py).start()
        rdma(full(py), full(py), fy_own_s, fy_own_r, py).wait_recv()
        rdma(bot(py), bot(py), hx_own_s, hx_own_r, px).start()
        for k in range(KMAX):
            @pl.when(z >= k + 1)
            def _(k=k):
                o = px - 4 * (k + 1)
                rdma(full(o), full(o), fx_up_s.at[k], fx_up_r.at[k], px).wait_recv()
                rdma(top(o), top(o), gy_up_s.at[k], gy_up_r.at[k], py).start()
                o = py - 4 * (k + 1)
                rdma(full(o), full(o), fy_up_s.at[k], fy_up_r.at[k], py).wait_recv()
                rdma(bot(o), bot(o), hx_up_s.at[k], hx_up_r.at[k], px).start()

            @pl.when(z <= 2 - k)
            def _(k=k):
                o = px + 4 * (k + 1)
                rdma(full(o), full(o), fx_dn_s.at[k], fx_dn_r.at[k], px).wait_recv()
                rdma(top(o), top(o), gy_dn_s.at[k], gy_dn_r.at[k], py).start()
                o = py + 4 * (k + 1)
                rdma(full(o), full(o), fy_dn_s.at[k], fy_dn_r.at[k], py).wait_recv()
                rdma(bot(o), bot(o), hx_dn_s.at[k], hx_dn_r.at[k], px).start()

        dg = 4 * z + (3 - jnp.bitwise_xor(s, 1))
        rdma(bot(dg), bot(dg), hx_own_s, hx_own_r, px).wait_recv()
        rdma(top(dg), top(dg), gy_own_s, gy_own_r, py).wait_recv()
        for k in range(KMAX):
            @pl.when(z >= k + 1)
            def _(k=k):
                o = dg - 4 * (k + 1)
                rdma(bot(o), bot(o), hx_up_s.at[k], hx_up_r.at[k], px).wait_recv()
                rdma(top(o), top(o), gy_up_s.at[k], gy_up_r.at[k], py).wait_recv()

            @pl.when(z <= 2 - k)
            def _(k=k):
                o = dg + 4 * (k + 1)
                rdma(bot(o), bot(o), hx_dn_s.at[k], hx_dn_r.at[k], px).wait_recv()
                rdma(top(o), top(o), gy_dn_s.at[k], gy_dn_r.at[k], py).wait_recv()

        @pl.when(z < NZ - 1)
        def _():
            rdma(x_ref, full(my), up_s.at[0], up_r.at[0], up).wait_send()

        @pl.when(z > 0)
        def _():
            rdma(x_ref, full(my), dn_s.at[0], dn_r.at[0], dn).wait_send()

        rdma(x_ref, full(my), fx_own_s, fx_own_r, px).wait_send()
        rdma(x_ref, full(my), fy_own_s, fy_own_r, py).wait_send()
        rdma(top(dg), top(dg), gy_own_s, gy_own_r, py).wait_send()
        rdma(bot(dg), bot(dg), hx_own_s, hx_own_r, px).wait_send()
        for k in range(KMAX):
            @pl.when(z >= k + 1)
            def _(k=k):
                o = my - 4 * (k + 1)
                if k < KMAX - 1:
                    @pl.when(z < NZ - 1)
                    def _():
                        rdma(
                            full(o), full(o),
                            up_s.at[k + 1], up_r.at[k + 1], up,
                        ).wait_send()
                rdma(full(o), full(o), fx_up_s.at[k], fx_up_r.at[k], px).wait_send()
                rdma(full(o), full(o), fy_up_s.at[k], fy_up_r.at[k], py).wait_send()
                rdma(top(o), top(o), gy_up_s.at[k], gy_up_r.at[k], py).wait_send()
                rdma(bot(o), bot(o), hx_up_s.at[k], hx_up_r.at[k], px).wait_send()

            @pl.when(z <= 2 - k)
            def _(k=k):
                o = my + 4 * (k + 1)
                if k < KMAX - 1:
                    @pl.when(z > 0)
                    def _():
                        rdma(
                            full(o), full(o),
                            dn_s.at[k + 1], dn_r.at[k + 1], dn,
                        ).wait_send()
                rdma(full(o), full(o), fx_dn_s.at[k], fx_dn_r.at[k], px).wait_send()
                rdma(full(o), full(o), fy_dn_s.at[k], fy_dn_r.at[k], py).wait_send()
                rdma(top(o), top(o), gy_dn_s.at[k], gy_dn_r.at[k], py).wait_send()
                rdma(bot(o), bot(o), hx_dn_s.at[k], hx_dn_r.at[k], px).wait_send()

        local_copy.wait()

    dma = pltpu.SemaphoreType.DMA
    out, _ = pl.pallas_call(
        body,
        out_shape=[
            jax.ShapeDtypeStruct((N_DEV * m_per, n), jnp.bfloat16),
            jax.ShapeDtypeStruct((8, 128), jnp.bfloat16),
        ],
        in_specs=[pl.BlockSpec(memory_space=pltpu.VMEM)],
        out_specs=[
            pl.BlockSpec(memory_space=pl.ANY),
            pl.BlockSpec(memory_space=pl.ANY),
        ],
        scratch_shapes=[
            dma,
            dma((KMAX,)), dma((KMAX,)), dma((KMAX,)), dma((KMAX,)),
            dma, dma, dma, dma,
            dma((KMAX,)), dma((KMAX,)), dma((KMAX,)), dma((KMAX,)),
            dma((KMAX,)), dma((KMAX,)), dma((KMAX,)), dma((KMAX,)),
            dma, dma, dma, dma,
            dma((KMAX,)), dma((KMAX,)), dma((KMAX,)), dma((KMAX,)),
            dma((KMAX,)), dma((KMAX,)), dma((KMAX,)), dma((KMAX,)),
        ],
        compiler_params=pltpu.CompilerParams(collective_id=0),
    )(x16)
    return out
